# baseline (speedup 1.0000x reference)
import functools

import jax
import jax.numpy as jnp
from jax import lax
from jax.experimental import pallas as pl
from jax.experimental.pallas import tpu as pltpu
from jax.experimental.pallas import tpu_sc as plsc

NC = 2
NS = 16
LANES = 16


def _enc_body(x_ref, w_ref, b_ref, o_ref):
    o_ref[0] = (
        jnp.dot(x_ref[...], w_ref[0], preferred_element_type=jnp.float32)
        + b_ref[0]
    )


def _encoder(x, node_W, node_b, bn):
    n, d = x.shape
    nblk = pl.cdiv(n, bn)
    return pl.pallas_call(
        _enc_body,
        grid=(2, nblk),
        in_specs=[
            pl.BlockSpec((bn, d), lambda s, i: (i, 0)),
            pl.BlockSpec((1, d, d), lambda s, i: (s, 0, 0)),
            pl.BlockSpec((1, 1, d), lambda s, i: (s, 0, 0)),
        ],
        out_specs=pl.BlockSpec((1, bn, d), lambda s, i: (s, i, 0)),
        out_shape=jax.ShapeDtypeStruct((2, n, d), jnp.float32),
    )(x, node_W, node_b.reshape(2, 1, d))


def _eemb_body(a_ref, w_ref, b_ref, o_ref):
    o_ref[0] = (
        jnp.dot(a_ref[...], w_ref[0], preferred_element_type=jnp.float32)
        + b_ref[0]
    )


def _edge_embeddings(edge_attr, ew, eb, be):
    e, de = edge_attr.shape
    k, _, d = ew.shape
    nblk = pl.cdiv(e, be)
    return pl.pallas_call(
        _eemb_body,
        grid=(k, nblk),
        in_specs=[
            pl.BlockSpec((be, de), lambda j, i: (i, 0)),
            pl.BlockSpec((1, de, d), lambda j, i: (j, 0, 0)),
            pl.BlockSpec((1, 1, d), lambda j, i: (j, 0, 0)),
        ],
        out_specs=pl.BlockSpec((1, be, d), lambda j, i: (j, i, 0)),
        out_shape=jax.ShapeDtypeStruct((k, e, d), jnp.float32),
    )(edge_attr, ew, eb.reshape(k, 1, d))


def _make_upd_body(do_relu):
    def body(h_ref, g_ref, wh_ref, wa_ref, b_ref, m_ref, o_ref):
        ab = []
        for s in range(2):
            t = (
                jnp.dot(h_ref[s], wh_ref[s], preferred_element_type=jnp.float32)
                + jnp.dot(g_ref[s], wa_ref[s], preferred_element_type=jnp.float32)
                + b_ref[s]
            )
            if do_relu:
                t = jnp.maximum(t, 0.0)
            ab.append(t)
        o_ref[0] = m_ref[0, 0] * ab[0] + m_ref[0, 1] * ab[1]
        o_ref[1] = m_ref[1, 0] * ab[0] + m_ref[1, 1] * ab[1]
    return body


def _update(h, agg, wh, wa, bb, mm, do_relu, bn):
    _, n, d = h.shape
    nblk = pl.cdiv(n, bn)
    return pl.pallas_call(
        _make_upd_body(do_relu),
        grid=(nblk,),
        in_specs=[
            pl.BlockSpec((2, bn, d), lambda i: (0, i, 0)),
            pl.BlockSpec((2, bn, d), lambda i: (0, i, 0)),
            pl.BlockSpec((2, d, d), lambda i: (0, 0, 0)),
            pl.BlockSpec((2, d, d), lambda i: (0, 0, 0)),
            pl.BlockSpec((2, 1, d), lambda i: (0, 0, 0)),
            pl.BlockSpec(memory_space=pltpu.SMEM),
        ],
        out_specs=pl.BlockSpec((2, bn, d), lambda i: (0, i, 0)),
        out_shape=jax.ShapeDtypeStruct((2, n, d), jnp.float32),
    )(h, agg, wh, wa, bb.reshape(2, 1, d), mm)


def _make_sc_gin(n, e, d):
    cb = 40
    trip = e // (cb * NS)
    nps = n // NS
    zr = 25
    nz = nps // zr
    jpd = d // LANES

    def body(h_ref, e_ref, row_ref, col_ref, out_ref,
             row_v, col_v, e_v, h_v, msg_v,
             zero_v, agg_sh, sem_s, sem_g, sem_c):
        cid = lax.axis_index("c")
        sid = lax.axis_index("s")

        def zbody(i, _):
            for j in range(jpd):
                zero_v[i, pl.ds(j * LANES, LANES)] = jnp.zeros(
                    (LANES,), jnp.float32)
            return 0
        lax.fori_loop(0, zr, zbody, 0)
        for t in range(nz):
            r0 = sid * nps + t * zr
            pltpu.sync_copy(zero_v, agg_sh.at[pl.ds(r0, zr)])
        plsc.subcore_barrier()

        def bases(t):
            ch = sid + t * NS
            base = pl.multiple_of(ch * cb, cb)
            ebase = pl.multiple_of(cid * e + base, cb)
            return base, ebase

        def issue_small(t, p, pc):
            base, ebase = bases(t)
            pltpu.async_copy(row_ref.at[pl.ds(ebase, cb)], row_v[p], sem_s[p])
            pltpu.async_copy(col_ref.at[pl.ds(base, cb)], col_v[pc], sem_s[p])
            pltpu.async_copy(e_ref.at[pl.ds(ebase, cb)], e_v[p], sem_s[p])

        def wait_small(t, p, pc):
            base, ebase = bases(t)
            pltpu.make_async_copy(
                row_ref.at[pl.ds(ebase, cb)], row_v[p], sem_s[p]).wait()
            pltpu.make_async_copy(
                col_ref.at[pl.ds(base, cb)], col_v[pc], sem_s[p]).wait()
            pltpu.make_async_copy(
                e_ref.at[pl.ds(ebase, cb)], e_v[p], sem_s[p]).wait()

        def issue_gather(p):
            pltpu.async_copy(h_ref.at[row_v[p]], h_v[p], sem_g[p])

        def wait_gather(p):
            pltpu.make_async_copy(h_ref.at[row_v[p]], h_v[p], sem_g[p]).wait()

        def compute(p):
            def edge_body(i, _):
                for j in range(jpd):
                    sl = pl.ds(j * LANES, LANES)
                    msg_v[p][i, sl] = jnp.maximum(
                        h_v[p][i, sl] + e_v[p][i, sl], 0.0)
                return 0
            lax.fori_loop(0, cb, edge_body, 0, unroll=2)

        def issue_scatter(p, pc):
            pltpu.async_copy(msg_v[p], agg_sh.at[col_v[pc]], sem_c[p],
                             add=True)

        def wait_scatter(p, pc):
            pltpu.make_async_copy(msg_v[p], agg_sh.at[col_v[pc]],
                                  sem_c[p]).wait()

        issue_small(0, 0, 0)
        issue_small(1, 1, 1)
        wait_small(0, 0, 0)
        issue_gather(0)
        half = trip // 4 - 1

        def pipe_body(tt, _):
            for b in range(4):
                t = 4 * tt + b
                p = b % 2
                q = 1 - p
                wait_gather(p)
                if b < 2:
                    @pl.when(tt > 0)
                    def _():
                        wait_scatter(p, (b + 2) % 4)
                else:
                    wait_scatter(p, (b + 2) % 4)
                compute(p)
                issue_scatter(p, b)
                if b < 2:
                    issue_small(t + 2, p, (b + 2) % 4)
                else:
                    @pl.when(tt < half)
                    def _():
                        issue_small(t + 2, p, (b + 2) % 4)
                if b < 3:
                    wait_small(t + 1, q, (b + 1) % 4)
                    issue_gather(q)
                else:
                    @pl.when(tt < half)
                    def _():
                        wait_small(t + 1, q, 0)
                        issue_gather(q)
            return 0
        lax.fori_loop(0, trip // 4, pipe_body, 0)
        wait_scatter(0, 2)
        wait_scatter(1, 3)

        plsc.subcore_barrier()
        s8 = -(-n // (NS * 8)) * 8
        r0 = pl.multiple_of(sid * s8, 8)
        rows_last = n - s8 * (NS - 1)

        @pl.when(sid < NS - 1)
        def _():
            pltpu.sync_copy(agg_sh.at[pl.ds(r0, s8)],
                            out_ref.at[cid, pl.ds(r0, s8)])

        @pl.when(sid == NS - 1)
        def _():
            rl = pl.multiple_of((NS - 1) * s8, 8)
            pltpu.sync_copy(agg_sh.at[pl.ds(rl, rows_last)],
                            out_ref.at[cid, pl.ds(rl, rows_last)])

    return pl.kernel(
        body,
        out_type=jax.ShapeDtypeStruct((2, n, d), jnp.float32),
        mesh=plsc.VectorSubcoreMesh(core_axis_name="c", subcore_axis_name="s",
                                    num_cores=NC, num_subcores=NS),
        scratch_types=[
            [pltpu.VMEM((cb,), jnp.int32) for _ in range(2)],
            [pltpu.VMEM((cb,), jnp.int32) for _ in range(4)],
            [pltpu.VMEM((cb, d), jnp.float32) for _ in range(2)],
            [pltpu.VMEM((cb, d), jnp.float32) for _ in range(2)],
            [pltpu.VMEM((cb, d), jnp.float32) for _ in range(2)],
            pltpu.VMEM((zr, d), jnp.float32),
            pltpu.VMEM_SHARED((n, d), jnp.float32),
            [pltpu.SemaphoreType.DMA for _ in range(2)],
            [pltpu.SemaphoreType.DMA for _ in range(2)],
            [pltpu.SemaphoreType.DMA for _ in range(2)],
        ],
    )


def kernel(x, edge_index, edge_attr, node_W, node_b, lin_W, lin_b,
           edge_W, edge_b, eps, bn_gamma, bn_beta, cross):
    n, d = x.shape
    e = edge_index.shape[1]
    nl = lin_W.shape[0]

    row = edge_index[0]
    col = edge_index[1]
    row_off = jnp.concatenate([row, row + n])

    bn_inv = 1.0 / jnp.sqrt(1.0 + 1e-5)
    g = bn_gamma * bn_inv
    wa = lin_W * g[:, :, None, :]
    wh = wa * (1.0 + eps)[:, :, None, None]
    bb = lin_b * g + bn_beta
    m00 = cross[:, 0, 0]
    m01 = cross[:, 0, 1]
    m10 = cross[:, 1, 0] * m00
    m11 = cross[:, 1, 0] * m01 + cross[:, 1, 1]
    mm = jnp.stack([jnp.stack([m00, m01], -1),
                    jnp.stack([m10, m11], -1)], 1)

    h = _encoder(x, node_W, node_b, bn=1000)
    e_all = _edge_embeddings(
        edge_attr, edge_W.reshape(nl * 2, -1, d),
        edge_b.reshape(nl * 2, d), be=2000)

    sc_gin = _make_sc_gin(n, e, d)
    for l in range(nl):
        e_l = e_all[2 * l:2 * l + 2].reshape(2 * e, d)
        agg = sc_gin(h.reshape(2 * n, d), e_l, row_off, col)
        h = _update(h, agg, wh[l], wa[l], bb[l], mm[l],
                    do_relu=(l < nl - 1), bn=1000)
    return (h[0], h[1])


# speedup vs baseline: 1.5249x; 1.5249x over previous
import functools

import jax
import jax.numpy as jnp
from jax import lax
from jax.experimental import pallas as pl
from jax.experimental.pallas import tpu as pltpu
from jax.experimental.pallas import tpu_sc as plsc

NC = 2
NS = 16
LANES = 16


def _pack_half(x):
    hd = x.shape[-1] // 2
    ai = jax.lax.bitcast_convert_type(x, jnp.int32) + jnp.int32(0x8000)
    return (ai[:, hd:] & jnp.int32(-65536)) | (
        (ai[:, :hd] >> 16) & jnp.int32(0xFFFF))


def _enc_body(x_ref, w_ref, b_ref, o_ref):
    o_ref[0] = (
        jnp.dot(x_ref[...], w_ref[0], preferred_element_type=jnp.float32)
        + b_ref[0]
    )


def _encoder(x, node_W, node_b, bn):
    n, d = x.shape
    nblk = pl.cdiv(n, bn)
    return pl.pallas_call(
        _enc_body,
        grid=(2, nblk),
        in_specs=[
            pl.BlockSpec((bn, d), lambda s, i: (i, 0)),
            pl.BlockSpec((1, d, d), lambda s, i: (s, 0, 0)),
            pl.BlockSpec((1, 1, d), lambda s, i: (s, 0, 0)),
        ],
        out_specs=pl.BlockSpec((1, bn, d), lambda s, i: (s, i, 0)),
        out_shape=jax.ShapeDtypeStruct((2, n, d), jnp.float32),
    )(x, node_W, node_b.reshape(2, 1, d))


def _eemb_body(a_ref, w_ref, b_ref, o_ref):
    o_ref[0] = _pack_half(
        jnp.dot(a_ref[...], w_ref[0], preferred_element_type=jnp.float32)
        + b_ref[0]
    )


def _edge_embeddings(edge_attr, ew, eb, be):
    e, de = edge_attr.shape
    k, _, d = ew.shape
    nblk = pl.cdiv(e, be)
    return pl.pallas_call(
        _eemb_body,
        grid=(k, nblk),
        in_specs=[
            pl.BlockSpec((be, de), lambda j, i: (i, 0)),
            pl.BlockSpec((1, de, d), lambda j, i: (j, 0, 0)),
            pl.BlockSpec((1, 1, d), lambda j, i: (j, 0, 0)),
        ],
        out_specs=pl.BlockSpec((1, be, d // 2), lambda j, i: (j, i, 0)),
        out_shape=jax.ShapeDtypeStruct((k, e, d // 2), jnp.int32),
    )(edge_attr, ew, eb.reshape(k, 1, d))


def _make_upd_body(do_relu):
    def body(h_ref, g_ref, wh_ref, wa_ref, b_ref, m_ref, o_ref):
        ab = []
        for s in range(2):
            t = (
                jnp.dot(h_ref[s], wh_ref[s], preferred_element_type=jnp.float32)
                + jnp.dot(g_ref[s], wa_ref[s], preferred_element_type=jnp.float32)
                + b_ref[s]
            )
            if do_relu:
                t = jnp.maximum(t, 0.0)
            ab.append(t)
        o_ref[0] = m_ref[0, 0] * ab[0] + m_ref[0, 1] * ab[1]
        o_ref[1] = m_ref[1, 0] * ab[0] + m_ref[1, 1] * ab[1]
    return body


def _update(h, agg, wh, wa, bb, mm, do_relu, bn):
    _, n, d = h.shape
    nblk = pl.cdiv(n, bn)
    return pl.pallas_call(
        _make_upd_body(do_relu),
        grid=(nblk,),
        in_specs=[
            pl.BlockSpec((2, bn, d), lambda i: (0, i, 0)),
            pl.BlockSpec((2, bn, d), lambda i: (0, i, 0)),
            pl.BlockSpec((2, d, d), lambda i: (0, 0, 0)),
            pl.BlockSpec((2, d, d), lambda i: (0, 0, 0)),
            pl.BlockSpec((2, 1, d), lambda i: (0, 0, 0)),
            pl.BlockSpec(memory_space=pltpu.SMEM),
        ],
        out_specs=pl.BlockSpec((2, bn, d), lambda i: (0, i, 0)),
        out_shape=jax.ShapeDtypeStruct((2, n, d), jnp.float32),
    )(h, agg, wh, wa, bb.reshape(2, 1, d), mm)


def _make_sc_gin(n, e, d):
    cb = 40
    trip = e // (cb * NS)
    nps = n // NS
    zr = 25
    nz = nps // zr
    jpd = d // LANES
    hd = d // 2

    def body(h_ref, e_ref, row_ref, col_ref, out_ref,
             row_v, col_v, e_v, h_v, agg_sh, sem_s, sem_g, sem_c):
        cid = lax.axis_index("c")
        sid = lax.axis_index("s")

        def zbody(i, _):
            for j in range(jpd):
                h_v[0][i, pl.ds(j * LANES, LANES)] = jnp.zeros(
                    (LANES,), jnp.float32)
            return 0
        lax.fori_loop(0, zr, zbody, 0)
        for t in range(nz):
            r0 = sid * nps + t * zr
            pltpu.sync_copy(h_v[0].at[pl.ds(0, zr)], agg_sh.at[pl.ds(r0, zr)])
        plsc.subcore_barrier()

        def bases(t):
            ch = sid + t * NS
            base = pl.multiple_of(ch * cb, cb)
            ebase = pl.multiple_of(cid * e + base, cb)
            return base, ebase

        def issue_small(t, r):
            base, ebase = bases(t)
            pltpu.async_copy(row_ref.at[pl.ds(ebase, cb)], row_v[r], sem_s[r])
            pltpu.async_copy(col_ref.at[pl.ds(base, cb)], col_v[r], sem_s[r])
            pltpu.async_copy(e_ref.at[pl.ds(ebase, cb)], e_v[r], sem_s[r])

        def wait_small(t, r):
            base, ebase = bases(t)
            pltpu.make_async_copy(
                row_ref.at[pl.ds(ebase, cb)], row_v[r], sem_s[r]).wait()
            pltpu.make_async_copy(
                col_ref.at[pl.ds(base, cb)], col_v[r], sem_s[r]).wait()
            pltpu.make_async_copy(
                e_ref.at[pl.ds(ebase, cb)], e_v[r], sem_s[r]).wait()

        def issue_gather(r, p):
            pltpu.async_copy(h_ref.at[row_v[r]], h_v[p], sem_g[p])

        def wait_gather(r, p):
            pltpu.make_async_copy(h_ref.at[row_v[r]], h_v[p],
                                  sem_g[p]).wait()

        def compute(r, p):
            def edge_body(i, _):
                for jj in range(hd // LANES):
                    sl = pl.ds(jj * LANES, LANES)
                    sh = pl.ds(hd + jj * LANES, LANES)
                    ew = e_v[r][i, sl]
                    el = jax.lax.bitcast_convert_type(ew << 16, jnp.float32)
                    eh = jax.lax.bitcast_convert_type(
                        ew & jnp.int32(-65536), jnp.float32)
                    h_v[p][i, sl] = jnp.maximum(h_v[p][i, sl] + el, 0.0)
                    h_v[p][i, sh] = jnp.maximum(h_v[p][i, sh] + eh, 0.0)
                return 0
            lax.fori_loop(0, cb, edge_body, 0, unroll=2)

        def issue_scatter(r, p):
            pltpu.async_copy(h_v[p], agg_sh.at[col_v[r]], sem_c[p], add=True)

        def wait_scatter(r, p):
            pltpu.make_async_copy(h_v[p], agg_sh.at[col_v[r]],
                                  sem_c[p]).wait()

        issue_small(0, 0)
        issue_small(1, 1)
        issue_small(2, 2)
        wait_small(0, 0)
        issue_gather(0, 0)

        wait_small(1, 1)
        issue_gather(1, 1)
        issue_small(3, 3)
        wait_gather(0, 0)
        compute(0, 0)
        issue_scatter(0, 0)
        wait_small(2, 2)
        wait_scatter(0, 0)
        issue_gather(2, 0)
        issue_small(4, 0)
        wait_gather(1, 1)
        compute(1, 1)
        issue_scatter(1, 1)
        wait_small(3, 3)
        wait_scatter(1, 1)
        issue_gather(3, 1)
        issue_small(5, 1)
        wait_gather(2, 0)
        compute(2, 0)
        issue_scatter(2, 0)
        wait_small(4, 0)
        wait_scatter(2, 0)
        issue_gather(0, 0)
        issue_small(6, 2)
        wait_gather(3, 1)
        compute(3, 1)
        issue_scatter(3, 1)

        def pipe_body(tt, _):
            for b in range(4):
                t = 4 * tt + b + 4
                r = b
                p = b % 2
                r1 = (b + 1) % 4
                p1 = 1 - p

                @pl.when(t < trip - 1)
                def _():
                    wait_small(t + 1, r1)
                wait_scatter(r1, p1)

                @pl.when(t < trip - 1)
                def _():
                    issue_gather(r1, p1)

                @pl.when(t < trip - 3)
                def _():
                    issue_small(t + 3, (b + 3) % 4)
                wait_gather(r, p)
                compute(r, p)
                issue_scatter(r, p)
            return 0
        lax.fori_loop(0, (trip - 4) // 4, pipe_body, 0)
        wait_scatter((trip - 1) % 4, (trip - 1) % 2)

        plsc.subcore_barrier()
        s8 = -(-n // (NS * 8)) * 8
        r0 = pl.multiple_of(sid * s8, 8)
        rows_last = n - s8 * (NS - 1)

        @pl.when(sid < NS - 1)
        def _():
            pltpu.sync_copy(agg_sh.at[pl.ds(r0, s8)],
                            out_ref.at[cid, pl.ds(r0, s8)])

        @pl.when(sid == NS - 1)
        def _():
            rl = pl.multiple_of((NS - 1) * s8, 8)
            pltpu.sync_copy(agg_sh.at[pl.ds(rl, rows_last)],
                            out_ref.at[cid, pl.ds(rl, rows_last)])

    return pl.kernel(
        body,
        out_type=jax.ShapeDtypeStruct((2, n, d), jnp.float32),
        mesh=plsc.VectorSubcoreMesh(core_axis_name="c", subcore_axis_name="s",
                                    num_cores=NC, num_subcores=NS),
        scratch_types=[
            [pltpu.VMEM((cb,), jnp.int32) for _ in range(4)],
            [pltpu.VMEM((cb,), jnp.int32) for _ in range(4)],
            [pltpu.VMEM((cb, hd), jnp.int32) for _ in range(4)],
            [pltpu.VMEM((cb, d), jnp.float32) for _ in range(2)],
            pltpu.VMEM_SHARED((n, d), jnp.float32),
            [pltpu.SemaphoreType.DMA for _ in range(4)],
            [pltpu.SemaphoreType.DMA for _ in range(2)],
            [pltpu.SemaphoreType.DMA for _ in range(2)],
        ],
    )


def kernel(x, edge_index, edge_attr, node_W, node_b, lin_W, lin_b,
           edge_W, edge_b, eps, bn_gamma, bn_beta, cross):
    n, d = x.shape
    e = edge_index.shape[1]
    nl = lin_W.shape[0]

    row = edge_index[0]
    col = edge_index[1]
    row_off = jnp.concatenate([row, row + n])

    bn_inv = 1.0 / jnp.sqrt(1.0 + 1e-5)
    g = bn_gamma * bn_inv
    wa = lin_W * g[:, :, None, :]
    wh = wa * (1.0 + eps)[:, :, None, None]
    bb = lin_b * g + bn_beta
    m00 = cross[:, 0, 0]
    m01 = cross[:, 0, 1]
    m10 = cross[:, 1, 0] * m00
    m11 = cross[:, 1, 0] * m01 + cross[:, 1, 1]
    mm = jnp.stack([jnp.stack([m00, m01], -1),
                    jnp.stack([m10, m11], -1)], 1)

    h = _encoder(x, node_W, node_b, bn=1000)
    e_all = _edge_embeddings(
        edge_attr, edge_W.reshape(nl * 2, -1, d),
        edge_b.reshape(nl * 2, d), be=2000)

    sc_gin = _make_sc_gin(n, e, d)
    for l in range(nl):
        e_l = e_all[2 * l:2 * l + 2].reshape(2 * e, d // 2)
        agg = sc_gin(h.reshape(2 * n, d), e_l, row_off, col)
        h = _update(h, agg, wh[l], wa[l], bb[l], mm[l],
                    do_relu=(l < nl - 1), bn=1000)
    return (h[0], h[1])
